# Initial kernel scaffold; baseline (speedup 1.0000x reference)
import jax
import jax.numpy as jnp
from jax.experimental import pallas as pl


def kernel(h, e, edge_index, U_w, U_b, V_w, V_b, A_w, A_b, B_w, B_b, C_w, C_b, ln_h_g, ln_h_b, ln_e_g, ln_e_b):
    raise NotImplementedError("write your pallas kernel here")


# speedup vs baseline: 2.0541x; 2.0541x over previous
import functools

import jax
import jax.numpy as jnp
from jax import lax
from jax.experimental import pallas as pl
from jax.experimental.pallas import tpu as pltpu
from jax.experimental.pallas import tpu_sc as plsc

F32 = jnp.float32

_NC = 2
_NS = 16
_NW = _NC * _NS


def _node_proj_body(x_ref, w_ref, b_ref, uh_ref, av_ref, bh_ref):
    r = jnp.dot(x_ref[...], w_ref[...], preferred_element_type=F32) + b_ref[...]
    uh_ref[...] = r[:, :128]
    av_ref[...] = r[:, 128:384]
    bh_ref[...] = r[:, 384:]


def _edge_proj_body(e_ref, w_ref, b_ref, out_ref):
    out_ref[...] = (
        jnp.dot(e_ref[...], w_ref[...], preferred_element_type=F32) + b_ref[...]
    )


def _ln_block(x, g, b, eps=1e-5):
    m = jnp.mean(x, axis=-1, keepdims=True)
    v = jnp.mean(jnp.square(x - m), axis=-1, keepdims=True)
    return (x - m) / jnp.sqrt(v + eps) * g + b


def _edge_elem_body(avd_ref, bhs_ref, ce_ref, e_ref, g_ref, b_ref,
                    msg_ref, eout_ref):
    avd = avd_ref[...]
    a = avd[:, :128]
    v = avd[:, 128:]
    en = a + bhs_ref[...] + ce_ref[...]
    gates = jax.nn.sigmoid(en)
    msg_ref[...] = gates * v
    e_norm = _ln_block(en, g_ref[...], b_ref[...])
    eout_ref[...] = e_ref[...] + jnp.maximum(e_norm, 0.0)


def _node_finish_body(h_ref, uh_ref, p0_ref, p1_ref, g_ref, b_ref, out_ref):
    s = uh_ref[...] + p0_ref[...] + p1_ref[...]
    h_new = _ln_block(s, g_ref[...], b_ref[...])
    out_ref[...] = h_ref[...] + jnp.maximum(h_new, 0.0)


def _sc_mesh():
    return plsc.VectorSubcoreMesh(
        core_axis_name="c", subcore_axis_name="s",
        num_cores=_NC, num_subcores=_NS)


def _make_gather(N, E, K):
    EW = E // _NW
    CH = EW // K
    mesh = _sc_mesh()

    @functools.partial(
        pl.kernel,
        out_type=[
            jax.ShapeDtypeStruct((E, 256), F32),
            jax.ShapeDtypeStruct((E, 128), F32),
        ],
        mesh=mesh,
        scratch_types=[
            pltpu.VMEM((K,), jnp.int32),
            pltpu.VMEM((K,), jnp.int32),
            pltpu.VMEM((K, 256), F32),
            pltpu.VMEM((K, 128), F32),
            pltpu.SemaphoreType.DMA,
            pltpu.SemaphoreType.DMA,
        ],
        compiler_params=pltpu.CompilerParams(use_tc_tiling_on_sc=False),
    )
    def gather(av_hbm, bh_hbm, dst_hbm, src_hbm, avd_hbm, bhs_hbm,
               dstv, srcv, avrows, brows, sem_a, sem_b):
        cid = lax.axis_index("c")
        sid = lax.axis_index("s")
        wid = sid * _NC + cid
        base = wid * EW

        def body(c, carry):
            off = base + c * K
            pltpu.sync_copy(dst_hbm.at[pl.ds(off, K)], dstv)
            pltpu.sync_copy(src_hbm.at[pl.ds(off, K)], srcv)
            cp_a = pltpu.async_copy(av_hbm.at[dstv], avrows, sem_a)
            cp_b = pltpu.async_copy(bh_hbm.at[srcv], brows, sem_b)
            cp_a.wait()
            cp_b.wait()
            pltpu.sync_copy(avrows, avd_hbm.at[pl.ds(off, K)])
            pltpu.sync_copy(brows, bhs_hbm.at[pl.ds(off, K)])
            return carry

        lax.fori_loop(0, CH, body, 0)

    return gather


def _make_scatter(N_pad, E, K):
    EW = E // _NW
    CH = EW // K
    HALF = N_pad // 2
    NPT = HALF // _NS
    mesh = _sc_mesh()

    @functools.partial(
        pl.kernel,
        out_type=[
            jax.ShapeDtypeStruct((N_pad, 128), F32),
            jax.ShapeDtypeStruct((N_pad, 128), F32),
        ],
        mesh=mesh,
        scratch_types=[
            pltpu.VMEM_SHARED((HALF + 8, 128), F32),
            pltpu.VMEM((K,), jnp.int32),
            pltpu.VMEM((K,), jnp.int32),
            pltpu.VMEM((K, 128), F32),
            pltpu.VMEM((NPT, 128), F32),
        ],
        compiler_params=pltpu.CompilerParams(use_tc_tiling_on_sc=False),
    )
    def scatter(msg_hbm, src_hbm, zeros_hbm, p0_hbm, p1_hbm,
                agg_sh, srcv, idxv, mrows, obuf):
        cid = lax.axis_index("c")
        sid = lax.axis_index("s")
        wid = sid * _NC + cid
        base = wid * EW
        trash = jnp.full((16,), HALF, jnp.int32)

        for p in range(2):
            lo = p * HALF

            @pl.when(sid == 0)
            def _():
                pltpu.sync_copy(zeros_hbm, agg_sh)

            plsc.subcore_barrier()

            def body(c, carry):
                off = base + c * K
                pltpu.sync_copy(src_hbm.at[pl.ds(off, K)], srcv)
                pltpu.sync_copy(msg_hbm.at[pl.ds(off, K)], mrows)
                for j in range(K // 16):
                    v = srcv[pl.ds(j * 16, 16)] - lo
                    ok = (v >= 0) & (v < HALF)
                    idxv[pl.ds(j * 16, 16)] = jnp.where(ok, v, trash)
                pltpu.sync_copy(mrows, agg_sh.at[idxv], add=True)
                return carry

            lax.fori_loop(0, CH, body, 0)
            plsc.subcore_barrier()

            rows = sid * NPT
            pltpu.sync_copy(agg_sh.at[pl.ds(rows, NPT)], obuf)

            @pl.when(cid == 0)
            def _():
                pltpu.sync_copy(obuf, p0_hbm.at[pl.ds(lo + rows, NPT)])

            @pl.when(cid == 1)
            def _():
                pltpu.sync_copy(obuf, p1_hbm.at[pl.ds(lo + rows, NPT)])

            plsc.subcore_barrier()

    return scatter


def kernel(h, e, edge_index, U_w, U_b, V_w, V_b, A_w, A_b, B_w, B_b, C_w, C_b,
           ln_h_g, ln_h_b, ln_e_g, ln_e_b):
    N, H = h.shape
    E = e.shape[0]
    src = edge_index[0]
    dst = edge_index[1]

    BN = 1000
    BE = 2000
    K = 80

    w_all = jnp.concatenate(
        [U_w.T, A_w.T, V_w.T, B_w.T], axis=1)
    b_all = jnp.concatenate([U_b, A_b, V_b, B_b]).reshape(1, 512)

    uh, av, bh = pl.pallas_call(
        _node_proj_body,
        grid=(N // BN,),
        in_specs=[
            pl.BlockSpec((BN, H), lambda i: (i, 0)),
            pl.BlockSpec((H, 4 * H), lambda i: (0, 0)),
            pl.BlockSpec((1, 4 * H), lambda i: (0, 0)),
        ],
        out_specs=[
            pl.BlockSpec((BN, H), lambda i: (i, 0)),
            pl.BlockSpec((BN, 2 * H), lambda i: (i, 0)),
            pl.BlockSpec((BN, H), lambda i: (i, 0)),
        ],
        out_shape=[
            jax.ShapeDtypeStruct((N, H), F32),
            jax.ShapeDtypeStruct((N, 2 * H), F32),
            jax.ShapeDtypeStruct((N, H), F32),
        ],
    )(h, w_all, b_all)

    ce = pl.pallas_call(
        _edge_proj_body,
        grid=(E // BE,),
        in_specs=[
            pl.BlockSpec((BE, H), lambda i: (i, 0)),
            pl.BlockSpec((H, H), lambda i: (0, 0)),
            pl.BlockSpec((1, H), lambda i: (0, 0)),
        ],
        out_specs=pl.BlockSpec((BE, H), lambda i: (i, 0)),
        out_shape=jax.ShapeDtypeStruct((E, H), F32),
    )(e, C_w.T, C_b.reshape(1, H))

    avd, bhs = _make_gather(N, E, K)(av, bh, dst, src)

    msg, e_out = pl.pallas_call(
        _edge_elem_body,
        grid=(E // BE,),
        in_specs=[
            pl.BlockSpec((BE, 2 * H), lambda i: (i, 0)),
            pl.BlockSpec((BE, H), lambda i: (i, 0)),
            pl.BlockSpec((BE, H), lambda i: (i, 0)),
            pl.BlockSpec((BE, H), lambda i: (i, 0)),
            pl.BlockSpec((1, H), lambda i: (0, 0)),
            pl.BlockSpec((1, H), lambda i: (0, 0)),
        ],
        out_specs=[
            pl.BlockSpec((BE, H), lambda i: (i, 0)),
            pl.BlockSpec((BE, H), lambda i: (i, 0)),
        ],
        out_shape=[
            jax.ShapeDtypeStruct((E, H), F32),
            jax.ShapeDtypeStruct((E, H), F32),
        ],
    )(avd, bhs, ce, e, ln_e_g.reshape(1, H), ln_e_b.reshape(1, H))

    N_pad = 10240
    zeros = jnp.zeros((N_pad // 2 + 8, H), F32)
    p0, p1 = _make_scatter(N_pad, E, K)(msg, src, zeros)

    h_out = pl.pallas_call(
        _node_finish_body,
        grid=(N // BN,),
        in_specs=[
            pl.BlockSpec((BN, H), lambda i: (i, 0)),
            pl.BlockSpec((BN, H), lambda i: (i, 0)),
            pl.BlockSpec((BN, H), lambda i: (i, 0)),
            pl.BlockSpec((BN, H), lambda i: (i, 0)),
            pl.BlockSpec((1, H), lambda i: (0, 0)),
            pl.BlockSpec((1, H), lambda i: (0, 0)),
        ],
        out_specs=pl.BlockSpec((BN, H), lambda i: (i, 0)),
        out_shape=jax.ShapeDtypeStruct((N, H), F32),
    )(h, uh, p0, p1, ln_h_g.reshape(1, H), ln_h_b.reshape(1, H))

    return (h_out, e_out)
